# baseline (speedup 1.0000x reference)
import jax
import jax.numpy as jnp
from jax.experimental import pallas as pl

B, L, D = 4, 4096, 2048
BL = 128


def _add_kernel(x_ref, emb_ref, o_ref):
    o_ref[...] = x_ref[...] + emb_ref[...][None, :, :]


def kernel(x, emb_weight):
    nl = L // BL
    return pl.pallas_call(
        _add_kernel,
        grid=(nl,),
        in_specs=[
            pl.BlockSpec((B, BL, D), lambda l: (0, l, 0)),
            pl.BlockSpec((BL, D), lambda l: (l, 0)),
        ],
        out_specs=pl.BlockSpec((B, BL, D), lambda l: (0, l, 0)),
        out_shape=jax.ShapeDtypeStruct((B, L, D), x.dtype),
    )(x, emb_weight)


# speedup vs baseline: 1.0058x; 1.0058x over previous
import jax
import jax.numpy as jnp
from jax.experimental import pallas as pl

B, L, D = 4, 4096, 2048
BL = 256


def _add_kernel(x_ref, emb_ref, o_ref):
    o_ref[...] = x_ref[...] + emb_ref[...][None, :, :]


def kernel(x, emb_weight):
    nl = L // BL
    return pl.pallas_call(
        _add_kernel,
        grid=(nl,),
        in_specs=[
            pl.BlockSpec((B, BL, D), lambda l: (0, l, 0)),
            pl.BlockSpec((BL, D), lambda l: (l, 0)),
        ],
        out_specs=pl.BlockSpec((B, BL, D), lambda l: (0, l, 0)),
        out_shape=jax.ShapeDtypeStruct((B, L, D), x.dtype),
    )(x, emb_weight)
